# Initial kernel scaffold; baseline (speedup 1.0000x reference)
import jax
import jax.numpy as jnp
from jax.experimental import pallas as pl


def kernel(x, gate_w, w1, w2, w3):
    raise NotImplementedError("write your pallas kernel here")


# speedup vs baseline: 1.4444x; 1.4444x over previous
import functools

import jax
import jax.numpy as jnp
from jax import lax
from jax.experimental import pallas as pl
from jax.experimental.pallas import tpu as pltpu
from jax.experimental.pallas import tpu_sc as plsc

E = 8
TOP_K = 2
T = 2048
D = 1024
H = 2816

B = 512
NB = (T * TOP_K + E * B) // B
P = NB * B
BH = 1408
NJ = H // BH

NW = 32
TPW = T // NW
CC = 32


def _routing_body(x_ref, gwt_ref, idx_ref, w_ref, be_ref):
    x = x_ref[...]
    gwt = gwt_ref[...]
    logits = jnp.dot(x, gwt, preferred_element_type=jnp.float32,
                     precision=jax.lax.Precision.DEFAULT)
    m = jnp.max(logits, axis=1, keepdims=True)
    p = jnp.exp(logits - m)
    sc = p / jnp.sum(p, axis=1, keepdims=True)

    lane = lax.broadcasted_iota(jnp.int32, (T, E), 1)
    m1 = jnp.max(sc, axis=1, keepdims=True)
    i1 = jnp.min(jnp.where(sc >= m1, lane, E), axis=1, keepdims=True)
    sc2 = jnp.where(lane == i1, -jnp.inf, sc)
    m2 = jnp.max(sc2, axis=1, keepdims=True)
    i2 = jnp.min(jnp.where(sc2 >= m2, lane, E), axis=1, keepdims=True)
    den = m1 + m2 + 1e-20
    w0 = m1 / den
    w1 = m2 / den

    sel = (lane == i1) | (lane == i2)
    mask_bf = jnp.where(sel, 1.0, 0.0).astype(jnp.bfloat16)

    r_io = lax.broadcasted_iota(jnp.int32, (T, T), 0)
    c_io = lax.broadcasted_iota(jnp.int32, (T, T), 1)
    tri = jnp.where(r_io > c_io, 1.0, 0.0).astype(jnp.bfloat16)
    rank = jax.lax.dot_general(tri, mask_bf, (((1,), (0,)), ((), ())),
                               preferred_element_type=jnp.float32)
    rank_i = rank.astype(jnp.int32)

    counts = jnp.sum(jnp.where(sel, 1.0, 0.0), axis=0).astype(jnp.int32)
    padded = ((counts + (B - 1)) // B) * B
    e_r = lax.broadcasted_iota(jnp.int32, (E, E), 0)
    e_c = lax.broadcasted_iota(jnp.int32, (E, E), 1)
    off = jnp.sum(jnp.where(e_c < e_r, padded[None, :], 0), axis=1)
    ends = off + padded

    dest_te = off[None, :] + rank_i
    d0 = jnp.sum(jnp.where(lane == i1, dest_te, 0), axis=1, keepdims=True)
    d1 = jnp.sum(jnp.where(lane == i2, dest_te, 0), axis=1, keepdims=True)

    idx_ref[...] = jnp.where(lane == 0, d0, jnp.where(lane == 1, d1, 0))
    w_ref[...] = jnp.where(lane == 0, w0, jnp.where(lane == 1, w1, 0.0))

    b_io = lax.broadcasted_iota(jnp.int32, (NB, E), 0)
    be = jnp.sum((b_io * B >= ends[None, :]).astype(jnp.int32),
                 axis=1, keepdims=True)
    be = jnp.minimum(be, E - 1)
    be_ref[...] = jnp.broadcast_to(be, (NB, E))


def _routing(x2, gwt):
    return pl.pallas_call(
        _routing_body,
        out_shape=[
            jax.ShapeDtypeStruct((T, E), jnp.int32),
            jax.ShapeDtypeStruct((T, E), jnp.float32),
            jax.ShapeDtypeStruct((NB, E), jnp.int32),
        ],
    )(x2, gwt)


def _scatter_body(x_hbm, d0_hbm, d1_hbm, xs_hbm, rows_v, i0_v, i1_v, s0, s1):
    wid = lax.axis_index("s") * 2 + lax.axis_index("c")
    base = wid * TPW
    pltpu.sync_copy(x_hbm.at[pl.ds(base, TPW)], rows_v)
    pltpu.sync_copy(d0_hbm.at[pl.ds(base, TPW)], i0_v)
    pltpu.sync_copy(d1_hbm.at[pl.ds(base, TPW)], i1_v)
    c0 = pltpu.async_copy(rows_v, xs_hbm.at[i0_v], s0)
    c1 = pltpu.async_copy(rows_v, xs_hbm.at[i1_v], s1)
    c0.wait()
    c1.wait()


def _scatter(x2, d0, d1):
    mesh = plsc.VectorSubcoreMesh(core_axis_name="c", subcore_axis_name="s")
    f = functools.partial(
        pl.kernel,
        out_type=jax.ShapeDtypeStruct((P, D), jnp.float32),
        mesh=mesh,
        scratch_types=[
            pltpu.VMEM((TPW, D), jnp.float32),
            pltpu.VMEM((TPW,), jnp.int32),
            pltpu.VMEM((TPW,), jnp.int32),
            pltpu.SemaphoreType.DMA,
            pltpu.SemaphoreType.DMA,
        ],
    )(_scatter_body)
    return f(x2, d0, d1)


def _ffn_body(be_ref, xs_ref, w1_ref, w3_ref, w2_ref, o_ref):
    j = pl.program_id(1)
    xb = xs_ref[...].astype(jnp.bfloat16)
    w1b = w1_ref[...].astype(jnp.bfloat16)
    w3b = w3_ref[...].astype(jnp.bfloat16)
    h1 = jax.lax.dot_general(xb, w1b, (((1,), (1,)), ((), ())),
                             preferred_element_type=jnp.float32)
    h3 = jax.lax.dot_general(xb, w3b, (((1,), (1,)), ((), ())),
                             preferred_element_type=jnp.float32)
    act = (h1 * (1.0 / (1.0 + jnp.exp(-h1)))) * h3
    w2b = w2_ref[...].astype(jnp.bfloat16)
    part = jax.lax.dot_general(act.astype(jnp.bfloat16), w2b,
                               (((1,), (1,)), ((), ())),
                               preferred_element_type=jnp.float32)

    @pl.when(j == 0)
    def _():
        o_ref[...] = part

    @pl.when(j != 0)
    def _():
        o_ref[...] = o_ref[...] + part


def _ffn(be, xs, w1, w3, w2):
    grid_spec = pltpu.PrefetchScalarGridSpec(
        num_scalar_prefetch=1,
        grid=(NB, NJ),
        in_specs=[
            pl.BlockSpec((B, D), lambda i, j, be: (i, 0)),
            pl.BlockSpec((None, BH, D), lambda i, j, be: (be[i], j, 0)),
            pl.BlockSpec((None, BH, D), lambda i, j, be: (be[i], j, 0)),
            pl.BlockSpec((None, D, BH), lambda i, j, be: (be[i], 0, j)),
        ],
        out_specs=pl.BlockSpec((B, D), lambda i, j, be: (i, 0)),
    )
    return pl.pallas_call(
        _ffn_body,
        grid_spec=grid_spec,
        out_shape=jax.ShapeDtypeStruct((P, D), jnp.float32),
        compiler_params=pltpu.CompilerParams(
            dimension_semantics=("arbitrary", "arbitrary")),
    )(be, xs, w1, w3, w2)


def _combine_body(ys_hbm, d0_hbm, d1_hbm, w0_hbm, w1_hbm, y_hbm,
                  g0_v, g1_v, i0_v, i1_v, wv0_v, wv1_v, s0, s1):
    wid = lax.axis_index("s") * 2 + lax.axis_index("c")
    base = wid * TPW
    for c in range(TPW // CC):
        cb = base + c * CC
        pltpu.sync_copy(d0_hbm.at[pl.ds(cb, CC)], i0_v)
        pltpu.sync_copy(d1_hbm.at[pl.ds(cb, CC)], i1_v)
        pltpu.sync_copy(w0_hbm.at[pl.ds(cb, CC)], wv0_v)
        pltpu.sync_copy(w1_hbm.at[pl.ds(cb, CC)], wv1_v)
        c0 = pltpu.async_copy(ys_hbm.at[i0_v], g0_v, s0)
        c1 = pltpu.async_copy(ys_hbm.at[i1_v], g1_v, s1)
        c0.wait()
        c1.wait()

        def tok_body(tt, _):
            q = (tt // 16) * 16
            r = jnp.full((16,), tt % 16, jnp.int32)
            a0 = wv0_v[pl.ds(q, 16)]
            a1 = wv1_v[pl.ds(q, 16)]
            dn = lax.GatherDimensionNumbers(
                offset_dims=(), collapsed_slice_dims=(0,), start_index_map=(0,))
            b0 = lax.gather(a0, r[:, None], dn, slice_sizes=(1,),
                            mode=lax.GatherScatterMode.PROMISE_IN_BOUNDS)
            b1 = lax.gather(a1, r[:, None], dn, slice_sizes=(1,),
                            mode=lax.GatherScatterMode.PROMISE_IN_BOUNDS)

            def vec_body(v, __):
                sl = pl.ds(v * 16, 16)
                g0_v[tt, sl] = b0 * g0_v[tt, sl] + b1 * g1_v[tt, sl]
                return __

            return lax.fori_loop(0, D // 16, vec_body, _)

        lax.fori_loop(0, CC, tok_body, 0)
        pltpu.sync_copy(g0_v, y_hbm.at[pl.ds(cb, CC)])


def _combine(ys, d0, d1, wa0, wa1):
    mesh = plsc.VectorSubcoreMesh(core_axis_name="c", subcore_axis_name="s")
    f = functools.partial(
        pl.kernel,
        out_type=jax.ShapeDtypeStruct((T, D), jnp.float32),
        mesh=mesh,
        scratch_types=[
            pltpu.VMEM((CC, D), jnp.float32),
            pltpu.VMEM((CC, D), jnp.float32),
            pltpu.VMEM((CC,), jnp.int32),
            pltpu.VMEM((CC,), jnp.int32),
            pltpu.VMEM((CC,), jnp.float32),
            pltpu.VMEM((CC,), jnp.float32),
            pltpu.SemaphoreType.DMA,
            pltpu.SemaphoreType.DMA,
        ],
    )(_combine_body)
    return f(ys, d0, d1, wa0, wa1)


def kernel(x, gate_w, w1, w2, w3):
    bsz, seq, d = x.shape
    x2 = x.reshape(T, D)
    gwt = gate_w.T
    idx_arr, w_arr, be_arr = _routing(x2, gwt)
    d0 = idx_arr[:, 0]
    d1 = idx_arr[:, 1]
    wa0 = w_arr[:, 0]
    wa1 = w_arr[:, 1]
    be = be_arr[:, 0]
    xs = _scatter(x2, d0, d1)
    ys = _ffn(be, xs, w1, w3, w2)
    y = _combine(ys, d0, d1, wa0, wa1)
    return y.reshape(bsz, seq, d)


# speedup vs baseline: 1.8132x; 1.2554x over previous
import functools

import jax
import jax.numpy as jnp
from jax import lax
from jax.experimental import pallas as pl
from jax.experimental.pallas import tpu as pltpu
from jax.experimental.pallas import tpu_sc as plsc

E = 8
TOP_K = 2
T = 2048
D = 1024
H = 2816

B = 512
NB = (T * TOP_K + E * B) // B
P = NB * B
BH = 1408
NJ = H // BH

NW = 32
TPW = T // NW
CC = 32


def _routing_body(x_ref, gwt_ref, idx_ref, w_ref, be_ref, nv_ref):
    x = x_ref[...]
    gwt = gwt_ref[...]
    logits = jnp.dot(x, gwt, preferred_element_type=jnp.float32,
                     precision=jax.lax.Precision.DEFAULT)
    m = jnp.max(logits, axis=1, keepdims=True)
    p = jnp.exp(logits - m)
    sc = p / jnp.sum(p, axis=1, keepdims=True)

    lane = lax.broadcasted_iota(jnp.int32, (T, E), 1)
    m1 = jnp.max(sc, axis=1, keepdims=True)
    i1 = jnp.min(jnp.where(sc >= m1, lane, E), axis=1, keepdims=True)
    sc2 = jnp.where(lane == i1, -jnp.inf, sc)
    m2 = jnp.max(sc2, axis=1, keepdims=True)
    i2 = jnp.min(jnp.where(sc2 >= m2, lane, E), axis=1, keepdims=True)
    den = m1 + m2 + 1e-20
    w0 = m1 / den
    w1 = m2 / den

    sel = (lane == i1) | (lane == i2)
    mask_bf = jnp.where(sel, 1.0, 0.0).astype(jnp.bfloat16)

    r_io = lax.broadcasted_iota(jnp.int32, (T, T), 0)
    c_io = lax.broadcasted_iota(jnp.int32, (T, T), 1)
    tri = jnp.where(r_io > c_io, 1.0, 0.0).astype(jnp.bfloat16)
    rank = jax.lax.dot_general(tri, mask_bf, (((1,), (0,)), ((), ())),
                               preferred_element_type=jnp.float32)
    rank_i = rank.astype(jnp.int32)

    counts = jnp.sum(jnp.where(sel, 1.0, 0.0), axis=0).astype(jnp.int32)
    padded = ((counts + (B - 1)) // B) * B
    e_r = lax.broadcasted_iota(jnp.int32, (E, E), 0)
    e_c = lax.broadcasted_iota(jnp.int32, (E, E), 1)
    off = jnp.sum(jnp.where(e_c < e_r, padded[None, :], 0), axis=1)
    ends = off + padded

    dest_te = off[None, :] + rank_i
    d0 = jnp.sum(jnp.where(lane == i1, dest_te, 0), axis=1, keepdims=True)
    d1 = jnp.sum(jnp.where(lane == i2, dest_te, 0), axis=1, keepdims=True)

    idx_ref[...] = jnp.where(lane == 0, d0, jnp.where(lane == 1, d1, 0))
    w_ref[...] = jnp.where(lane == 0, w0, jnp.where(lane == 1, w1, 0.0))

    b_io = lax.broadcasted_iota(jnp.int32, (NB, E), 0)
    be = jnp.sum((b_io * B >= ends[None, :]).astype(jnp.int32),
                 axis=1, keepdims=True)
    be = jnp.minimum(be, E - 1)
    be_ref[...] = jnp.broadcast_to(be, (NB, E))
    nv_ref[...] = jnp.broadcast_to(
        jnp.sum(padded[None, :], axis=1, keepdims=True) // B, (1, E))


def _routing(x2, gwt):
    return pl.pallas_call(
        _routing_body,
        out_shape=[
            jax.ShapeDtypeStruct((T, E), jnp.int32),
            jax.ShapeDtypeStruct((T, E), jnp.float32),
            jax.ShapeDtypeStruct((NB, E), jnp.int32),
            jax.ShapeDtypeStruct((1, E), jnp.int32),
        ],
    )(x2, gwt)


def _scatter_body(x_hbm, d0_hbm, d1_hbm, xs_hbm, rows_v, i0_v, i1_v, s0, s1):
    wid = lax.axis_index("s") * 2 + lax.axis_index("c")
    base = wid * TPW
    pltpu.sync_copy(x_hbm.at[pl.ds(base, TPW)], rows_v)
    pltpu.sync_copy(d0_hbm.at[pl.ds(base, TPW)], i0_v)
    pltpu.sync_copy(d1_hbm.at[pl.ds(base, TPW)], i1_v)
    c0 = pltpu.async_copy(rows_v, xs_hbm.at[i0_v], s0)
    c1 = pltpu.async_copy(rows_v, xs_hbm.at[i1_v], s1)
    c0.wait()
    c1.wait()


def _scatter(x2, d0, d1):
    mesh = plsc.VectorSubcoreMesh(core_axis_name="c", subcore_axis_name="s")
    f = functools.partial(
        pl.kernel,
        out_type=jax.ShapeDtypeStruct((P, D), jnp.float32),
        mesh=mesh,
        scratch_types=[
            pltpu.VMEM((TPW, D), jnp.float32),
            pltpu.VMEM((TPW,), jnp.int32),
            pltpu.VMEM((TPW,), jnp.int32),
            pltpu.SemaphoreType.DMA,
            pltpu.SemaphoreType.DMA,
        ],
    )(_scatter_body)
    return f(x2, d0, d1)


def _ffn_body(be_ref, nv_ref, xs_ref, w1_ref, w3_ref, w2_ref, o_ref,
              w1s, w3s, w2s, sems):
    i = pl.program_id(0)
    j = pl.program_id(1)
    nv = nv_ref[0]
    s = i * NJ + j

    def _issue(e, jj, slot):
        pltpu.make_async_copy(
            w1_ref.at[e, pl.ds(jj * BH, BH)], w1s.at[slot], sems.at[0, slot]
        ).start()
        pltpu.make_async_copy(
            w3_ref.at[e, pl.ds(jj * BH, BH)], w3s.at[slot], sems.at[1, slot]
        ).start()
        pltpu.make_async_copy(
            w2_ref.at[e, :, pl.ds(jj * BH, BH)], w2s.at[slot], sems.at[2, slot]
        ).start()

    @pl.when(s == 0)
    def _():
        _issue(be_ref[0], 0, 0)

    i_n = (s + 1) // NJ
    j_n = lax.rem(s + 1, NJ)

    @pl.when(jnp.logical_and(s + 1 < NB * NJ, i_n < nv))
    def _():
        _issue(be_ref[i_n], j_n, lax.rem(s + 1, 2))

    @pl.when(i < nv)
    def _():
        slot = lax.rem(s, 2)
        pltpu.make_async_copy(w1s.at[slot], w1s.at[slot], sems.at[0, slot]).wait()
        pltpu.make_async_copy(w3s.at[slot], w3s.at[slot], sems.at[1, slot]).wait()
        pltpu.make_async_copy(w2s.at[slot], w2s.at[slot], sems.at[2, slot]).wait()
        xb = xs_ref[...].astype(jnp.bfloat16)
        w1b = w1s[slot].astype(jnp.bfloat16)
        w3b = w3s[slot].astype(jnp.bfloat16)
        h1 = jax.lax.dot_general(xb, w1b, (((1,), (1,)), ((), ())),
                                 preferred_element_type=jnp.float32)
        h3 = jax.lax.dot_general(xb, w3b, (((1,), (1,)), ((), ())),
                                 preferred_element_type=jnp.float32)
        act = (h1 * (1.0 / (1.0 + jnp.exp(-h1)))) * h3
        w2b = w2s[slot].astype(jnp.bfloat16)
        part = jax.lax.dot_general(act.astype(jnp.bfloat16), w2b,
                                   (((1,), (1,)), ((), ())),
                                   preferred_element_type=jnp.float32)

        @pl.when(j == 0)
        def _():
            o_ref[...] = part

        @pl.when(j != 0)
        def _():
            o_ref[...] = o_ref[...] + part


def _ffn(be, nv, xs, w1, w3, w2):
    def _xs_map(i, j, be, nv):
        return (jnp.minimum(i, nv[0] - 1), 0)

    grid_spec = pltpu.PrefetchScalarGridSpec(
        num_scalar_prefetch=2,
        grid=(NB, NJ),
        in_specs=[
            pl.BlockSpec((B, D), _xs_map),
            pl.BlockSpec(memory_space=pl.ANY),
            pl.BlockSpec(memory_space=pl.ANY),
            pl.BlockSpec(memory_space=pl.ANY),
        ],
        out_specs=pl.BlockSpec((B, D), _xs_map),
        scratch_shapes=[
            pltpu.VMEM((2, BH, D), jnp.float32),
            pltpu.VMEM((2, BH, D), jnp.float32),
            pltpu.VMEM((2, D, BH), jnp.float32),
            pltpu.SemaphoreType.DMA((3, 2)),
        ],
    )
    return pl.pallas_call(
        _ffn_body,
        grid_spec=grid_spec,
        out_shape=jax.ShapeDtypeStruct((P, D), jnp.float32),
        compiler_params=pltpu.CompilerParams(
            dimension_semantics=("arbitrary", "arbitrary")),
    )(be, nv, xs, w1, w3, w2)


def _combine_body(ys_hbm, d0_hbm, d1_hbm, w0_hbm, w1_hbm, y_hbm,
                  g0_v, g1_v, i0_v, i1_v, wv0_v, wv1_v, s0, s1):
    wid = lax.axis_index("s") * 2 + lax.axis_index("c")
    base = wid * TPW
    for c in range(TPW // CC):
        cb = base + c * CC
        pltpu.sync_copy(d0_hbm.at[pl.ds(cb, CC)], i0_v)
        pltpu.sync_copy(d1_hbm.at[pl.ds(cb, CC)], i1_v)
        pltpu.sync_copy(w0_hbm.at[pl.ds(cb, CC)], wv0_v)
        pltpu.sync_copy(w1_hbm.at[pl.ds(cb, CC)], wv1_v)
        c0 = pltpu.async_copy(ys_hbm.at[i0_v], g0_v, s0)
        c1 = pltpu.async_copy(ys_hbm.at[i1_v], g1_v, s1)
        c0.wait()
        c1.wait()

        def tok_body(tt, _):
            q = (tt // 16) * 16
            r = jnp.full((16,), tt % 16, jnp.int32)
            a0 = wv0_v[pl.ds(q, 16)]
            a1 = wv1_v[pl.ds(q, 16)]
            dn = lax.GatherDimensionNumbers(
                offset_dims=(), collapsed_slice_dims=(0,), start_index_map=(0,))
            b0 = lax.gather(a0, r[:, None], dn, slice_sizes=(1,),
                            mode=lax.GatherScatterMode.PROMISE_IN_BOUNDS)
            b1 = lax.gather(a1, r[:, None], dn, slice_sizes=(1,),
                            mode=lax.GatherScatterMode.PROMISE_IN_BOUNDS)

            def vec_body(v, __):
                sl = pl.ds(v * 16, 16)
                g0_v[tt, sl] = b0 * g0_v[tt, sl] + b1 * g1_v[tt, sl]
                return __

            return lax.fori_loop(0, D // 16, vec_body, _)

        lax.fori_loop(0, CC, tok_body, 0)
        pltpu.sync_copy(g0_v, y_hbm.at[pl.ds(cb, CC)])


def _combine(ys, d0, d1, wa0, wa1):
    mesh = plsc.VectorSubcoreMesh(core_axis_name="c", subcore_axis_name="s")
    f = functools.partial(
        pl.kernel,
        out_type=jax.ShapeDtypeStruct((T, D), jnp.float32),
        mesh=mesh,
        scratch_types=[
            pltpu.VMEM((CC, D), jnp.float32),
            pltpu.VMEM((CC, D), jnp.float32),
            pltpu.VMEM((CC,), jnp.int32),
            pltpu.VMEM((CC,), jnp.int32),
            pltpu.VMEM((CC,), jnp.float32),
            pltpu.VMEM((CC,), jnp.float32),
            pltpu.SemaphoreType.DMA,
            pltpu.SemaphoreType.DMA,
        ],
    )(_combine_body)
    return f(ys, d0, d1, wa0, wa1)


def kernel(x, gate_w, w1, w2, w3):
    bsz, seq, d = x.shape
    x2 = x.reshape(T, D)
    gwt = gate_w.T
    idx_arr, w_arr, be_arr, nv_arr = _routing(x2, gwt)
    d0 = idx_arr[:, 0]
    d1 = idx_arr[:, 1]
    wa0 = w_arr[:, 0]
    wa1 = w_arr[:, 1]
    be = be_arr[:, 0]
    nv = nv_arr[0, :1]
    xs = _scatter(x2, d0, d1)
    ys = _ffn(be, nv, xs, w1, w3, w2)
    y = _combine(ys, d0, d1, wa0, wa1)
    return y.reshape(bsz, seq, d)
